# baseline (speedup 1.0000x reference)
import functools

import jax
import jax.numpy as jnp
from jax import lax
from jax.experimental import pallas as pl
from jax.experimental.pallas import tpu as pltpu
from jax.experimental.pallas import tpu_sc as plsc

R = 512
C = 100000
J = 16384
L = 16
NC = 2
NS = 16
NW = NC * NS
Z = 80
NSLAB = 39
PCAP = 3200
VROWS = 24
BIG = 0x40000000

_mesh = plsc.VectorSubcoreMesh(core_axis_name="c", subcore_axis_name="s",
                               num_cores=NC, num_subcores=NS)
_scratch = [
    pltpu.VMEM((Z, R), jnp.float32),
    pltpu.VMEM((Z, R), jnp.float32),
    pltpu.VMEM((J,), jnp.int32),
    pltpu.VMEM((PCAP,), jnp.int32),
    pltpu.VMEM((2 * VROWS * R,), jnp.float32),
    pltpu.VMEM((L,), jnp.int32),
    pltpu.SMEM((2 * VROWS,), jnp.int32),
    pltpu.SemaphoreType.DMA,
    pltpu.SemaphoreType.DMA,
    pltpu.SemaphoreType.DMA,
    pltpu.SemaphoreType.DMA,
    pltpu.SemaphoreType.DMA,
    pltpu.SemaphoreType.DMA,
]


def _sc_body(tT, idx_hbm, vflat, outT, bufa, bufb, idxbuf, pbuf, vrows,
             scr16, lrsmem, sia, sib, soa, sob, svra, svrb):
    wid = lax.axis_index("s") * NC + lax.axis_index("c")
    iota = lax.broadcasted_iota(jnp.int32, (L,), 0)
    q0 = (12500 * wid) >> 5
    q1 = (12500 * (wid + 1)) >> 5
    base = 8 * q0
    nrows = 8 * (q1 - q0)
    has8 = nrows == 3128
    colvecs = [iota + 16 * k for k in range(R // L)]

    pltpu.sync_copy(idx_hbm, idxbuf)
    perms = [lax.rem(iota + k, L) for k in range(1, L)]
    laters = [p > iota for p in perms]

    def dedup_body(g, carry):
        grp = idxbuf[pl.ds(g * L, L)]
        scr16[...] = grp
        loser = iota < 0
        for p, lat in zip(perms, laters):
            rot = plsc.load_gather(scr16, [p])
            loser = loser | ((rot == grp) & lat)
        idxbuf[pl.ds(g * L, L)] = jnp.where(loser, BIG, grp)
        return carry

    lax.fori_loop(0, J // L, dedup_body, 0)

    neg1 = jnp.full((L,), -1, jnp.int32)

    def pinit(g, carry):
        pbuf[pl.ds(g * L, L)] = neg1
        return carry

    lax.fori_loop(0, PCAP // L, pinit, 0)

    def pbuild(g, carry):
        ids = idxbuf[pl.ds(g * L, L)]
        lr = ids - base
        ok = (lr >= 0) & (lr < nrows)
        safe = jnp.where(ok, lr, 0)
        plsc.store_scatter(pbuf, [safe], g * L + iota, mask=ok)
        return carry

    lax.fori_loop(0, J // L, pbuild, 0)

    def vsem(par):
        return svra if par == 0 else svrb

    def scan_issue(z0, ng, par, lo):
        vbase = par * VROWS * R

        def scan_grp(g, k):
            pv = pbuf[pl.ds(z0 + g * L, L)]
            for lane in range(L):
                pvl = pv[lane]
                hit = pvl >= 0
                inwin = hit & (k >= lo) & (k < lo + VROWS)

                @pl.when(inwin)
                def _(pvl=pvl, k=k, g=g, lane=lane):
                    slot = k - lo
                    lrsmem[par * VROWS + slot] = g * L + lane
                    bj = (pvl >> 3) * 4096 + (pvl & 7) * 128
                    dst0 = vbase + slot * R

                    def frag(cr, carry3):
                        pltpu.make_async_copy(
                            vflat.at[pl.ds(bj + cr * 1024, 128)],
                            vrows.at[pl.ds(dst0 + cr * 128, 128)],
                            vsem(par)).start()
                        return carry3

                    lax.fori_loop(0, 4, frag, 0)

                k = k + jnp.where(hit, 1, 0)
            return k

        lax.fori_loop(0, ng, scan_grp, jnp.int32(0))

    def count_hits(z0, ng):
        def cnt_grp(g, k):
            pv = pbuf[pl.ds(z0 + g * L, L)]
            return k + plsc.all_reduce_population_count(pv >= 0)[0]

        return lax.fori_loop(0, ng, cnt_grp, jnp.int32(0))

    def scan_fetch(z0, ng, par):
        cnt = count_hits(z0, ng)

        @pl.when(cnt > 0)
        def _():
            scan_issue(z0, ng, par, jnp.int32(0))

        return cnt

    def drain_copy(buf, par, cnt, lo):
        take = jnp.minimum(cnt - lo, VROWS)
        vbase = par * VROWS * R

        def drain(h, carry2):
            pltpu.make_async_copy(vflat.at[pl.ds(0, R)],
                                  vrows.at[pl.ds(0, R)], vsem(par)).wait()
            return carry2

        lax.fori_loop(0, take, drain, 0)

        def copy_row(h, carry2):
            lr = lrsmem[par * VROWS + h]
            rowvec = jnp.full((L,), 0, jnp.int32) + lr
            for k in range(R // L):
                x = vrows[pl.ds(vbase + h * R + 16 * k, L)]
                plsc.store_scatter(buf, [rowvec, colvecs[k]], x)
            return carry2

        lax.fori_loop(0, take, copy_row, 0)

    def merge_apply(z0, ng, buf, par, cnt):
        @pl.when(cnt > 0)
        def _():
            drain_copy(buf, par, cnt, jnp.int32(0))

        nb = (cnt + VROWS - 1) // VROWS

        def extra(b, carry):
            scan_issue(z0, ng, par, b * VROWS)
            drain_copy(buf, par, cnt, b * VROWS)
            return carry

        lax.fori_loop(1, nb, extra, 0)

    NGZ = Z // L

    def cp_in(s, buf, sem):
        return pltpu.make_async_copy(tT.at[pl.ds(base + s * Z, Z)], buf, sem)

    def cp_out(s, buf, sem):
        return pltpu.make_async_copy(buf, outT.at[pl.ds(base + s * Z, Z)],
                                     sem)

    cp_in(0, bufa, sia).start()
    cnt0 = scan_fetch(0, NGZ, 0)

    def pair_body(i, cnt_a):
        a = 2 * i
        b = a + 1

        @pl.when(i > 0)
        def _():
            cp_out(a - 1, bufb, sob).wait()

        cp_in(b, bufb, sib).start()
        cnt_b = scan_fetch(b * Z, NGZ, 1)
        cp_in(a, bufa, sia).wait()
        merge_apply(a * Z, NGZ, bufa, 0, cnt_a)
        cp_out(a, bufa, soa).start()

        cnt_a2 = scan_fetch((a + 2) * Z, NGZ, 0)
        cp_out(a, bufa, soa).wait()

        @pl.when(a + 2 < NSLAB)
        def _():
            cp_in(a + 2, bufa, sia).start()

        cp_in(b, bufb, sib).wait()
        merge_apply(b * Z, NGZ, bufb, 1, cnt_b)
        cp_out(b, bufb, sob).start()
        return cnt_a2

    cnt_last = lax.fori_loop(0, NSLAB // 2, pair_body, cnt0)

    s_last = NSLAB - 1
    cp_in(s_last, bufa, sia).wait()
    merge_apply(s_last * Z, NGZ, bufa, 0, cnt_last)
    cp_out(s_last, bufa, soa).start()
    cp_out(s_last - 1, bufb, sob).wait()

    @pl.when(has8)
    def _():
        z8 = NSLAB * Z
        cp8 = pltpu.make_async_copy(tT.at[pl.ds(base + z8, 8)],
                                    bufb.at[pl.ds(0, 8), pl.ds(0, R)], sib)
        cp8.start()
        cnt8 = scan_fetch(z8, 1, 1)
        cp8.wait()
        merge_apply(z8, 1, bufb, 1, cnt8)
        pltpu.sync_copy(bufb.at[pl.ds(0, 8), pl.ds(0, R)],
                        outT.at[pl.ds(base + z8, 8)])

    cp_out(s_last, bufa, soa).wait()


_sc_kernel = functools.partial(
    pl.kernel,
    out_type=jax.ShapeDtypeStruct((C, R), jnp.float32),
    mesh=_mesh,
    scratch_types=_scratch,
    compiler_params=pltpu.CompilerParams(needs_layout_passes=False),
)(_sc_body)


def kernel(t, idx, v):
    tT = jnp.transpose(t)
    vflat = v.reshape(4, 128, 2048, 8).transpose(2, 0, 3, 1).reshape(-1)
    outT = _sc_kernel(tT, idx, vflat)
    return jnp.transpose(outT)


# speedup vs baseline: 1.1593x; 1.1593x over previous
import functools

import jax
import jax.numpy as jnp
from jax import lax
from jax.experimental import pallas as pl
from jax.experimental.pallas import tpu as pltpu
from jax.experimental.pallas import tpu_sc as plsc

R = 512
C = 100000
J = 16384
L = 16
NC = 2
NS = 16
NW = NC * NS
Z = 80
NSLAB = 39
PCAP = 3200
VROWS = 24
BIG = 0x40000000

_mesh = plsc.VectorSubcoreMesh(core_axis_name="c", subcore_axis_name="s",
                               num_cores=NC, num_subcores=NS)
_scratch = [
    pltpu.VMEM((Z, R), jnp.float32),
    pltpu.VMEM((Z, R), jnp.float32),
    pltpu.VMEM((J,), jnp.int32),
    pltpu.VMEM((PCAP,), jnp.int32),
    pltpu.VMEM((2 * VROWS * R,), jnp.float32),
    pltpu.VMEM((L,), jnp.int32),
    pltpu.SMEM((2 * VROWS,), jnp.int32),
    pltpu.SemaphoreType.DMA,
    pltpu.SemaphoreType.DMA,
    pltpu.SemaphoreType.DMA,
    pltpu.SemaphoreType.DMA,
    pltpu.SemaphoreType.DMA,
    pltpu.SemaphoreType.DMA,
]


def _sc_body(tT, idx_hbm, vflat, outT, bufa, bufb, idxbuf, pbuf, vrows,
             scr16, lrsmem, sia, sib, soa, sob, svra, svrb):
    wid = lax.axis_index("s") * NC + lax.axis_index("c")
    iota = lax.broadcasted_iota(jnp.int32, (L,), 0)
    q0 = (12500 * wid) >> 5
    q1 = (12500 * (wid + 1)) >> 5
    base = 8 * q0
    nrows = 8 * (q1 - q0)
    has8 = nrows == 3128
    colvecs = [iota + 16 * k for k in range(R // L)]

    pltpu.sync_copy(idx_hbm, idxbuf)
    perms = [lax.rem(iota + k, L) for k in range(1, L)]
    laters = [p > iota for p in perms]

    def dedup_body(g, carry):
        grp = idxbuf[pl.ds(g * L, L)]
        scr16[...] = grp
        loser = iota < 0
        for p, lat in zip(perms, laters):
            rot = plsc.load_gather(scr16, [p])
            loser = loser | ((rot == grp) & lat)
        idxbuf[pl.ds(g * L, L)] = jnp.where(loser, BIG, grp)
        return carry

    lax.fori_loop(0, J // L, dedup_body, 0)

    neg1 = jnp.full((L,), -1, jnp.int32)

    def pinit(g, carry):
        pbuf[pl.ds(g * L, L)] = neg1
        return carry

    lax.fori_loop(0, PCAP // L, pinit, 0)

    def pbuild(g, carry):
        ids = idxbuf[pl.ds(g * L, L)]
        lr = ids - base
        ok = (lr >= 0) & (lr < nrows)
        safe = jnp.where(ok, lr, 0)
        plsc.store_scatter(pbuf, [safe], g * L + iota, mask=ok)
        return carry

    lax.fori_loop(0, J // L, pbuild, 0)

    def vsem(par):
        return svra if par == 0 else svrb

    def scan_issue(z0, ng, par, lo):
        vbase = par * VROWS * R

        def scan_grp(g, k):
            pv = pbuf[pl.ds(z0 + g * L, L)]
            for lane in range(L):
                pvl = pv[lane]
                hit = pvl >= 0
                inwin = hit & (k >= lo) & (k < lo + VROWS)

                @pl.when(inwin)
                def _(pvl=pvl, k=k, g=g, lane=lane):
                    slot = k - lo
                    lrsmem[par * VROWS + slot] = g * L + lane
                    pltpu.make_async_copy(
                        vflat.at[pl.ds(pvl * R, R)],
                        vrows.at[pl.ds(vbase + slot * R, R)],
                        vsem(par)).start()

                k = k + jnp.where(hit, 1, 0)
            return k

        lax.fori_loop(0, ng, scan_grp, jnp.int32(0))

    def count_hits(z0, ng):
        def cnt_grp(g, k):
            pv = pbuf[pl.ds(z0 + g * L, L)]
            return k + plsc.all_reduce_population_count(pv >= 0)[0]

        return lax.fori_loop(0, ng, cnt_grp, jnp.int32(0))

    def scan_fetch(z0, ng, par):
        cnt = count_hits(z0, ng)

        @pl.when(cnt > 0)
        def _():
            scan_issue(z0, ng, par, jnp.int32(0))

        return cnt

    def drain_copy(buf, par, cnt, lo):
        take = jnp.minimum(cnt - lo, VROWS)
        vbase = par * VROWS * R

        def drain(h, carry2):
            pltpu.make_async_copy(vflat.at[pl.ds(0, R)],
                                  vrows.at[pl.ds(0, R)], vsem(par)).wait()
            return carry2

        lax.fori_loop(0, take, drain, 0)

        def copy_row(h, carry2):
            lr = lrsmem[par * VROWS + h]
            rowvec = jnp.full((L,), 0, jnp.int32) + lr
            for k in range(R // L):
                x = vrows[pl.ds(vbase + h * R + 16 * k, L)]
                plsc.store_scatter(buf, [rowvec, colvecs[k]], x)
            return carry2

        lax.fori_loop(0, take, copy_row, 0)

    def merge_apply(z0, ng, buf, par, cnt):
        @pl.when(cnt > 0)
        def _():
            drain_copy(buf, par, cnt, jnp.int32(0))

        nb = (cnt + VROWS - 1) // VROWS

        def extra(b, carry):
            scan_issue(z0, ng, par, b * VROWS)
            drain_copy(buf, par, cnt, b * VROWS)
            return carry

        lax.fori_loop(1, nb, extra, 0)

    NGZ = Z // L

    def cp_in(s, buf, sem):
        return pltpu.make_async_copy(tT.at[pl.ds(base + s * Z, Z)], buf, sem)

    def cp_out(s, buf, sem):
        return pltpu.make_async_copy(buf, outT.at[pl.ds(base + s * Z, Z)],
                                     sem)

    cp_in(0, bufa, sia).start()
    cnt0 = scan_fetch(0, NGZ, 0)

    def pair_body(i, cnt_a):
        a = 2 * i
        b = a + 1

        @pl.when(i > 0)
        def _():
            cp_out(a - 1, bufb, sob).wait()

        cp_in(b, bufb, sib).start()
        cnt_b = scan_fetch(b * Z, NGZ, 1)
        cp_in(a, bufa, sia).wait()
        merge_apply(a * Z, NGZ, bufa, 0, cnt_a)
        cp_out(a, bufa, soa).start()

        cnt_a2 = scan_fetch((a + 2) * Z, NGZ, 0)
        cp_out(a, bufa, soa).wait()

        @pl.when(a + 2 < NSLAB)
        def _():
            cp_in(a + 2, bufa, sia).start()

        cp_in(b, bufb, sib).wait()
        merge_apply(b * Z, NGZ, bufb, 1, cnt_b)
        cp_out(b, bufb, sob).start()
        return cnt_a2

    cnt_last = lax.fori_loop(0, NSLAB // 2, pair_body, cnt0)

    s_last = NSLAB - 1
    cp_in(s_last, bufa, sia).wait()
    merge_apply(s_last * Z, NGZ, bufa, 0, cnt_last)
    cp_out(s_last, bufa, soa).start()
    cp_out(s_last - 1, bufb, sob).wait()

    @pl.when(has8)
    def _():
        z8 = NSLAB * Z
        cp8 = pltpu.make_async_copy(tT.at[pl.ds(base + z8, 8)],
                                    bufb.at[pl.ds(0, 8), pl.ds(0, R)], sib)
        cp8.start()
        cnt8 = scan_fetch(z8, 1, 1)
        cp8.wait()
        merge_apply(z8, 1, bufb, 1, cnt8)
        pltpu.sync_copy(bufb.at[pl.ds(0, 8), pl.ds(0, R)],
                        outT.at[pl.ds(base + z8, 8)])

    cp_out(s_last, bufa, soa).wait()


_sc_kernel = functools.partial(
    pl.kernel,
    out_type=jax.ShapeDtypeStruct((C, R), jnp.float32),
    mesh=_mesh,
    scratch_types=_scratch,
    compiler_params=pltpu.CompilerParams(needs_layout_passes=False),
)(_sc_body)


def kernel(t, idx, v):
    tT = jnp.transpose(t)
    vflat = jnp.transpose(v).reshape(-1)
    outT = _sc_kernel(tT, idx, vflat)
    return jnp.transpose(outT)


# speedup vs baseline: 1.2556x; 1.0830x over previous
import functools

import jax
import jax.numpy as jnp
from jax import lax
from jax.experimental import pallas as pl
from jax.experimental.pallas import tpu as pltpu
from jax.experimental.pallas import tpu_sc as plsc

R = 512
C = 100000
J = 16384
L = 16
NC = 2
NS = 16
NW = NC * NS
Z = 80
NSLAB = 39
PCAP = 3200
VROWS = 24
BIG = 0x40000000

_mesh = plsc.VectorSubcoreMesh(core_axis_name="c", subcore_axis_name="s",
                               num_cores=NC, num_subcores=NS)
_scratch = [
    pltpu.VMEM((Z, R), jnp.float32),
    pltpu.VMEM((Z, R), jnp.float32),
    pltpu.VMEM((J,), jnp.int32),
    pltpu.VMEM((PCAP,), jnp.int32),
    pltpu.VMEM((2 * VROWS * R,), jnp.float32),
    pltpu.VMEM((L,), jnp.int32),
    pltpu.SMEM((2 * VROWS,), jnp.int32),
    pltpu.VMEM_SHARED((J,), jnp.int32),
    pltpu.SemaphoreType.DMA,
    pltpu.SemaphoreType.DMA,
    pltpu.SemaphoreType.DMA,
    pltpu.SemaphoreType.DMA,
    pltpu.SemaphoreType.DMA,
    pltpu.SemaphoreType.DMA,
]


def _sc_body(tT, idx_hbm, vflat, outT, bufa, bufb, idxbuf, pbuf, vrows,
             scr16, lrsmem, shidx, sia, sib, soa, sob, svra, svrb):
    wid = lax.axis_index("s") * NC + lax.axis_index("c")
    iota = lax.broadcasted_iota(jnp.int32, (L,), 0)
    q0 = (12500 * wid) >> 5
    q1 = (12500 * (wid + 1)) >> 5
    base = 8 * q0
    nrows = 8 * (q1 - q0)
    has8 = nrows == 3128
    colvecs = [iota + 16 * k for k in range(R // L)]

    sid = lax.axis_index("s")
    gchunk = J // L // NS
    echunk = gchunk * L
    pltpu.sync_copy(idx_hbm.at[pl.ds(sid * echunk, echunk)],
                    idxbuf.at[pl.ds(sid * echunk, echunk)])
    perms = [lax.rem(iota + k, L) for k in range(1, L)]
    laters = [p > iota for p in perms]

    def dedup_body(g, carry):
        grp = idxbuf[pl.ds(g * L, L)]
        scr16[...] = grp
        loser = iota < 0
        for p, lat in zip(perms, laters):
            rot = plsc.load_gather(scr16, [p])
            loser = loser | ((rot == grp) & lat)
        idxbuf[pl.ds(g * L, L)] = jnp.where(loser, BIG, grp)
        return carry

    lax.fori_loop(sid * gchunk, (sid + 1) * gchunk, dedup_body, 0)
    pltpu.sync_copy(idxbuf.at[pl.ds(sid * echunk, echunk)],
                    shidx.at[pl.ds(sid * echunk, echunk)])
    plsc.subcore_barrier()
    pltpu.sync_copy(shidx, idxbuf)

    neg1 = jnp.full((L,), -1, jnp.int32)

    def pinit(g, carry):
        pbuf[pl.ds(g * L, L)] = neg1
        return carry

    lax.fori_loop(0, PCAP // L, pinit, 0)

    def pbuild(g, carry):
        ids = idxbuf[pl.ds(g * L, L)]
        lr = ids - base
        ok = (lr >= 0) & (lr < nrows)
        safe = jnp.where(ok, lr, 0)
        plsc.store_scatter(pbuf, [safe], g * L + iota, mask=ok)
        return carry

    lax.fori_loop(0, J // L, pbuild, 0)

    def vsem(par):
        return svra if par == 0 else svrb

    def scan_issue(z0, ng, par, lo):
        vbase = par * VROWS * R

        def scan_grp(g, k):
            pv = pbuf[pl.ds(z0 + g * L, L)]
            for lane in range(L):
                pvl = pv[lane]
                hit = pvl >= 0
                inwin = hit & (k >= lo) & (k < lo + VROWS)

                @pl.when(inwin)
                def _(pvl=pvl, k=k, g=g, lane=lane):
                    slot = k - lo
                    lrsmem[par * VROWS + slot] = g * L + lane
                    pltpu.make_async_copy(
                        vflat.at[pl.ds(pvl * R, R)],
                        vrows.at[pl.ds(vbase + slot * R, R)],
                        vsem(par)).start()

                k = k + jnp.where(hit, 1, 0)
            return k

        return lax.fori_loop(0, ng, scan_grp, jnp.int32(0))

    def scan_fetch(z0, ng, par):
        return scan_issue(z0, ng, par, jnp.int32(0))

    def drain_copy(buf, par, cnt, lo):
        take = jnp.minimum(cnt - lo, VROWS)
        vbase = par * VROWS * R

        def drain(h, carry2):
            pltpu.make_async_copy(vflat.at[pl.ds(0, R)],
                                  vrows.at[pl.ds(0, R)], vsem(par)).wait()
            return carry2

        lax.fori_loop(0, take, drain, 0)

        def copy_row(h, carry2):
            lr = lrsmem[par * VROWS + h]
            rowvec = jnp.full((L,), 0, jnp.int32) + lr
            for k in range(R // L):
                x = vrows[pl.ds(vbase + h * R + 16 * k, L)]
                plsc.store_scatter(buf, [rowvec, colvecs[k]], x)
            return carry2

        lax.fori_loop(0, take, copy_row, 0)

    def merge_apply(z0, ng, buf, par, cnt):
        @pl.when(cnt > 0)
        def _():
            drain_copy(buf, par, cnt, jnp.int32(0))

        nb = (cnt + VROWS - 1) // VROWS

        def extra(b, carry):
            scan_issue(z0, ng, par, b * VROWS)
            drain_copy(buf, par, cnt, b * VROWS)
            return carry

        lax.fori_loop(1, nb, extra, 0)

    NGZ = Z // L

    def cp_in(s, buf, sem):
        return pltpu.make_async_copy(tT.at[pl.ds(base + s * Z, Z)], buf, sem)

    def cp_out(s, buf, sem):
        return pltpu.make_async_copy(buf, outT.at[pl.ds(base + s * Z, Z)],
                                     sem)

    cp_in(0, bufa, sia).start()
    cnt0 = scan_fetch(0, NGZ, 0)

    def pair_body(i, cnt_a):
        a = 2 * i
        b = a + 1

        @pl.when(i > 0)
        def _():
            cp_out(a - 1, bufb, sob).wait()

        cp_in(b, bufb, sib).start()
        cnt_b = scan_fetch(b * Z, NGZ, 1)
        cp_in(a, bufa, sia).wait()
        merge_apply(a * Z, NGZ, bufa, 0, cnt_a)
        cp_out(a, bufa, soa).start()

        cnt_a2 = scan_fetch((a + 2) * Z, NGZ, 0)
        cp_out(a, bufa, soa).wait()

        @pl.when(a + 2 < NSLAB)
        def _():
            cp_in(a + 2, bufa, sia).start()

        cp_in(b, bufb, sib).wait()
        merge_apply(b * Z, NGZ, bufb, 1, cnt_b)
        cp_out(b, bufb, sob).start()
        return cnt_a2

    cnt_last = lax.fori_loop(0, NSLAB // 2, pair_body, cnt0)

    s_last = NSLAB - 1
    cp_in(s_last, bufa, sia).wait()
    merge_apply(s_last * Z, NGZ, bufa, 0, cnt_last)
    cp_out(s_last, bufa, soa).start()
    cp_out(s_last - 1, bufb, sob).wait()

    @pl.when(has8)
    def _():
        z8 = NSLAB * Z
        cp8 = pltpu.make_async_copy(tT.at[pl.ds(base + z8, 8)],
                                    bufb.at[pl.ds(0, 8), pl.ds(0, R)], sib)
        cp8.start()
        cnt8 = scan_fetch(z8, 1, 1)
        cp8.wait()
        merge_apply(z8, 1, bufb, 1, cnt8)
        pltpu.sync_copy(bufb.at[pl.ds(0, 8), pl.ds(0, R)],
                        outT.at[pl.ds(base + z8, 8)])

    cp_out(s_last, bufa, soa).wait()


_sc_kernel = functools.partial(
    pl.kernel,
    out_type=jax.ShapeDtypeStruct((C, R), jnp.float32),
    mesh=_mesh,
    scratch_types=_scratch,
    compiler_params=pltpu.CompilerParams(needs_layout_passes=False),
)(_sc_body)


def kernel(t, idx, v):
    tT = jnp.transpose(t)
    vflat = jnp.transpose(v).reshape(-1)
    outT = _sc_kernel(tT, idx, vflat)
    return jnp.transpose(outT)
